# baseline (speedup 1.0000x reference)
import jax
import jax.numpy as jnp
from jax import lax
from jax.experimental import pallas as pl
from jax.experimental.pallas import tpu as pltpu, tpu_sc as plsc

import functools


def kernel(x, tables):
    F, V, D = tables.shape
    B, F2 = x.shape
    assert F == F2

    info = plsc.get_sparse_core_info()
    NC, NS, L = info.num_cores, info.num_subcores, info.num_lanes
    NW = NC * NS
    N = B * F
    per_w = N // NW
    assert per_w * NW == N and per_w % F == 0 and per_w % L == 0
    GCH = 128
    n_g = per_w // GCH
    assert n_g * GCH == per_w

    flat_tables = tables.reshape(F * V, D)
    x_flat = (x.astype(jnp.int32) + jnp.arange(F, dtype=jnp.int32)[None, :] * V).reshape(N)

    mesh = plsc.VectorSubcoreMesh(core_axis_name="c", subcore_axis_name="s")

    @functools.partial(
        pl.kernel,
        mesh=mesh,
        compiler_params=pltpu.CompilerParams(use_tc_tiling_on_sc=False),
        out_type=jax.ShapeDtypeStruct((N, D), jnp.float32),
        scratch_types=[
            pltpu.VMEM((per_w,), jnp.int32),
            pltpu.VMEM((per_w, D), jnp.float32),
            pltpu.SemaphoreType.DMA,
            pltpu.SemaphoreType.DMA,
        ],
    )
    def emb_kernel(tab_hbm, idx_hbm, out_hbm, idx_v, rows_v, sem_i, sem_g):
        wid = lax.axis_index("s") * NC + lax.axis_index("c")
        base = wid * per_w

        pltpu.async_copy(idx_hbm.at[pl.ds(base, per_w)], idx_v, sem_i).wait()

        copies = [
            pltpu.make_async_copy(
                tab_hbm.at[idx_v.at[pl.ds(j * GCH, GCH)]],
                rows_v.at[pl.ds(j * GCH, GCH)],
                sem_g,
            )
            for j in range(n_g)
        ]
        for c in copies:
            c.start()
        for c in copies:
            c.wait()

        pltpu.async_copy(rows_v, out_hbm.at[pl.ds(base, per_w)], sem_i).wait()

    out_flat = emb_kernel(flat_tables, x_flat)
    return out_flat.reshape(B, F, D)


# speedup vs baseline: 5.1212x; 5.1212x over previous
import jax
import jax.numpy as jnp
from jax import lax
from jax.experimental import pallas as pl
from jax.experimental.pallas import tpu as pltpu, tpu_sc as plsc

import functools


def kernel(x, tables):
    F, V, D = tables.shape
    B, F2 = x.shape
    assert F == F2 and F <= 32

    info = plsc.get_sparse_core_info()
    NC, NS, L = info.num_cores, info.num_subcores, info.num_lanes
    NW = NC * NS
    b_per_w = B // NW
    assert b_per_w * NW == B

    mesh = plsc.VectorSubcoreMesh(core_axis_name="c", subcore_axis_name="s")

    @functools.partial(
        pl.kernel,
        mesh=mesh,
        compiler_params=pltpu.CompilerParams(use_tc_tiling_on_sc=True),
        out_type=jax.ShapeDtypeStruct((B, F, D), jnp.float32),
        scratch_types=[
            pltpu.VMEM((b_per_w, F), jnp.int32),
            pltpu.SemaphoreType.DMA,
            pltpu.SemaphoreType.DMA,
        ],
    )
    def emb_kernel(x_hbm, tab_hbm, out_hbm, idx_v, sem_x, sem_g):
        wid = lax.axis_index("s") * NC + lax.axis_index("c")
        b0w = wid * b_per_w

        pltpu.async_copy(x_hbm.at[pl.ds(b0w, b_per_w), :], idx_v, sem_x).wait()

        def enqueue(sb, _):
            b = b0w + sb
            v0 = idx_v[sb, pl.ds(0, L)]
            v1 = idx_v[sb, pl.ds(F - L, L)]
            for f in range(F):
                i = v0[f] if f < L else v1[f - (F - L)]
                pltpu.async_copy(
                    tab_hbm.at[pl.ds(f, 1), pl.ds(i, 1), :],
                    out_hbm.at[pl.ds(b, 1), pl.ds(f, 1), :],
                    sem_g,
                )
            return 0

        lax.fori_loop(0, b_per_w, enqueue, 0)

        def drain(k, _):
            pltpu.make_async_copy(
                tab_hbm.at[pl.ds(0, 1), pl.ds(0, 1), :],
                out_hbm.at[pl.ds(0, 1), pl.ds(0, 1), :],
                sem_g,
            ).wait()
            return 0

        lax.fori_loop(0, b_per_w * F, drain, 0)

    return emb_kernel(x, tables)


# speedup vs baseline: 14.7550x; 2.8812x over previous
import jax
import jax.numpy as jnp
from jax import lax
from jax.experimental import pallas as pl
from jax.experimental.pallas import tpu as pltpu, tpu_sc as plsc

import functools

_SB = 8


def kernel(x, tables):
    F, V, D = tables.shape
    B, F2 = x.shape
    assert F == F2 and F <= 32

    info = plsc.get_sparse_core_info()
    NC, NS, L = info.num_cores, info.num_subcores, info.num_lanes
    NW = NC * NS
    b_per_w = B // NW
    assert b_per_w * NW == B and b_per_w % _SB == 0
    n_sub = b_per_w // _SB

    mesh = plsc.VectorSubcoreMesh(core_axis_name="c", subcore_axis_name="s")

    @functools.partial(
        pl.kernel,
        mesh=mesh,
        compiler_params=pltpu.CompilerParams(use_tc_tiling_on_sc=True),
        out_type=jax.ShapeDtypeStruct((B, F, D), jnp.float32),
        scratch_types=[
            pltpu.VMEM((b_per_w, F), jnp.int32),
            pltpu.VMEM((2, _SB, F, D), jnp.float32),
            pltpu.SemaphoreType.DMA,
            pltpu.SemaphoreType.DMA,
            pltpu.SemaphoreType.DMA,
        ],
    )
    def emb_kernel(x_hbm, tab_hbm, out_hbm, idx_v, rows_v, sem_x, sem_g,
                   sem_w):
        wid = lax.axis_index("s") * NC + lax.axis_index("c")
        b0w = wid * b_per_w

        pltpu.async_copy(x_hbm.at[pl.ds(b0w, b_per_w), :], idx_v, sem_x).wait()

        def sub_chunk(s, _):
            r0 = s * _SB
            p = lax.rem(s, 2)
            buf = rows_v.at[p]

            @pl.when(s >= 2)
            def _():
                pltpu.make_async_copy(
                    rows_v.at[0], out_hbm.at[pl.ds(0, _SB), :, :], sem_w
                ).wait()

            def enqueue(sb, _):
                v0 = idx_v[r0 + sb, pl.ds(0, L)]
                v1 = idx_v[r0 + sb, pl.ds(F - L, L)]
                for f in range(F):
                    i = v0[f] if f < L else v1[f - (F - L)]
                    pltpu.async_copy(
                        tab_hbm.at[pl.ds(f, 1), pl.ds(i, 1), :],
                        buf.at[pl.ds(sb, 1), pl.ds(f, 1), :],
                        sem_g,
                    )
                return 0

            lax.fori_loop(0, _SB, enqueue, 0)

            def drain(k, _):
                pltpu.make_async_copy(
                    tab_hbm.at[pl.ds(0, 1), pl.ds(0, 1), :],
                    rows_v.at[0, pl.ds(0, 1), pl.ds(0, 1), :],
                    sem_g,
                ).wait()
                return 0

            lax.fori_loop(0, _SB * F, drain, 0)

            pltpu.make_async_copy(
                buf, out_hbm.at[pl.ds(b0w + r0, _SB), :, :], sem_w
            ).start()
            return 0

        lax.fori_loop(0, n_sub, sub_chunk, 0)

        def drain_wb(k, _):
            pltpu.make_async_copy(
                rows_v.at[0], out_hbm.at[pl.ds(0, _SB), :, :], sem_w
            ).wait()
            return 0

        lax.fori_loop(0, 2, drain_wb, 0)

    return emb_kernel(x, tables)


# speedup vs baseline: 60.6046x; 4.1074x over previous
import jax
import jax.numpy as jnp
from jax import lax
from jax.experimental import pallas as pl
from jax.experimental.pallas import tpu as pltpu, tpu_sc as plsc

import functools


def kernel(x, tables):
    F, V, D = tables.shape
    B, F2 = x.shape
    assert F == F2

    info = plsc.get_sparse_core_info()
    NC, NS, L = info.num_cores, info.num_subcores, info.num_lanes
    NW = NC * NS
    P = F * D
    p_per_w = P // NW
    assert p_per_w * NW == P and B % L == 0

    tab_t = tables.transpose(0, 2, 1)
    x_t = x.T

    mesh = plsc.VectorSubcoreMesh(core_axis_name="c", subcore_axis_name="s")

    @functools.partial(
        pl.kernel,
        mesh=mesh,
        compiler_params=pltpu.CompilerParams(use_tc_tiling_on_sc=True, needs_layout_passes=False),
        out_type=jax.ShapeDtypeStruct((F, D, B), jnp.float32),
        scratch_types=[
            pltpu.VMEM((2, B), jnp.int32),
            pltpu.VMEM((V,), jnp.float32),
            pltpu.VMEM((2, 1, B), jnp.float32),
            pltpu.SemaphoreType.DMA,
            pltpu.SemaphoreType.DMA,
            pltpu.SemaphoreType.DMA,
        ],
    )
    def emb_kernel(x_hbm, tab_hbm, out_hbm, idx_v, col_v, outp_v, sem_i,
                   sem_c, sem_w):
        wid = lax.axis_index("s") * NC + lax.axis_index("c")
        p0 = wid * p_per_w
        f_lo = lax.shift_right_logical(p0, 5)

        pltpu.async_copy(x_hbm.at[pl.ds(f_lo, 1), :],
                         idx_v.at[pl.ds(0, 1), :], sem_i).wait()
        f_hi = jnp.minimum(f_lo + 1, F - 1)
        pltpu.async_copy(x_hbm.at[pl.ds(f_hi, 1), :],
                         idx_v.at[pl.ds(1, 1), :], sem_i).wait()

        for k in range(p_per_w):
            p = p0 + k
            f = lax.shift_right_logical(p, 5)
            d = lax.bitwise_and(p, D - 1)
            fi = f - f_lo
            pb = k % 2

            pltpu.async_copy(tab_hbm.at[f, d, :], col_v, sem_c).wait()

            if k >= 2:
                pltpu.make_async_copy(
                    outp_v.at[pl.ds(pb, 1)],
                    out_hbm.at[pl.ds(0, 1), pl.ds(0, 1), :],
                    sem_w,
                ).wait()

            def gather(g, _):
                s = pl.ds(g * L, L)
                idx16 = idx_v[fi, s]
                outp_v[pb, 0, s] = plsc.load_gather(col_v, [idx16])
                return 0

            lax.fori_loop(0, B // L, gather, 0)

            pltpu.make_async_copy(
                outp_v.at[pl.ds(pb, 1)],
                out_hbm.at[pl.ds(f, 1), pl.ds(d, 1), :],
                sem_w,
            ).start()

        for _ in range(2):
            pltpu.make_async_copy(
                outp_v.at[pl.ds(0, 1)],
                out_hbm.at[pl.ds(0, 1), pl.ds(0, 1), :],
                sem_w,
            ).wait()

    out_t = emb_kernel(x_t, tab_t)
    return out_t.transpose(2, 0, 1)
